# Initial kernel scaffold; baseline (speedup 1.0000x reference)
import jax
import jax.numpy as jnp
from jax.experimental import pallas as pl


def kernel(hidden_states, gate_weight, ws, w2s):
    raise NotImplementedError("write your pallas kernel here")


# speedup vs baseline: 1.9663x; 1.9663x over previous
import functools

import jax
import jax.numpy as jnp
from jax.experimental import pallas as pl
from jax.experimental.pallas import tpu as pltpu

T = 1024
H = 2048
I = 2048
E = 8
JITTER_EPS = 0.01
BI = 512
NI = I // BI


def _routing_body(scores_ref, comb_ref):
    scores = scores_ref[...]
    col = jax.lax.broadcasted_iota(jnp.int32, scores.shape, 1)
    ninf = jnp.float32(-jnp.inf)

    m1 = jnp.max(scores, axis=1, keepdims=True)
    ind1 = jnp.min(jnp.where(scores == m1, col, E), axis=1, keepdims=True)
    factor1 = jnp.maximum(jnp.abs(scores), m1)
    mask1 = ((m1 - scores) / factor1) > (2.0 * JITTER_EPS)
    mg1 = jnp.where(mask1, ninf, scores)
    e1 = jnp.exp(mg1 - m1)
    p1 = e1 / jnp.sum(e1, axis=1, keepdims=True)
    mult1 = jnp.sum(jnp.where(col == ind1, p1, 0.0), axis=1, keepdims=True)

    masked_scores = jnp.where(col == ind1, ninf, scores)
    m2 = jnp.max(masked_scores, axis=1, keepdims=True)
    ind2 = jnp.min(jnp.where(masked_scores == m2, col, E), axis=1, keepdims=True)
    factor2 = jnp.maximum(jnp.abs(scores), m2)
    mask2 = ((m2 - scores) / factor2) > (2.0 * JITTER_EPS)
    mg2 = jnp.where(mask2, ninf, masked_scores)
    e2 = jnp.exp(mg2 - m2)
    p2 = e2 / jnp.sum(e2, axis=1, keepdims=True)
    mult2 = jnp.sum(jnp.where(col == ind2, p2, 0.0), axis=1, keepdims=True)

    comb_ref[...] = (jnp.where(col == ind1, mult1, 0.0)
                     + jnp.where(col == ind2, mult2, 0.0))


def _routing(scores):
    return pl.pallas_call(
        _routing_body,
        out_shape=jax.ShapeDtypeStruct((T, E), jnp.float32),
    )(scores)


def _ffn_body(x_ref, w1_ref, w3_ref, w2_ref, comb_ref, out_ref):
    e = pl.program_id(0)
    i = pl.program_id(1)

    @pl.when((e == 0) & (i == 0))
    def _init():
        out_ref[...] = jnp.zeros_like(out_ref)

    x = x_ref[...]
    w1 = w1_ref[0].astype(jnp.bfloat16)
    w3 = w3_ref[0].astype(jnp.bfloat16)
    h1 = jax.lax.dot_general(x, w1, (((1,), (1,)), ((), ())),
                             preferred_element_type=jnp.float32)
    h3 = jax.lax.dot_general(x, w3, (((1,), (1,)), ((), ())),
                             preferred_element_type=jnp.float32)
    act = (h1 * jax.nn.sigmoid(h1) * h3).astype(jnp.bfloat16)
    w2 = w2_ref[0].astype(jnp.bfloat16)
    y = jax.lax.dot_general(act, w2, (((1,), (1,)), ((), ())),
                            preferred_element_type=jnp.float32)
    comb = comb_ref[...]
    ecol = jax.lax.broadcasted_iota(jnp.int32, comb.shape, 1)
    cvec = jnp.sum(jnp.where(ecol == e, comb, 0.0), axis=1, keepdims=True)
    out_ref[...] += cvec * y


def _ffn(x_bf16, ws, w2s, comb):
    grid = (E, NI)
    return pl.pallas_call(
        _ffn_body,
        grid=grid,
        in_specs=[
            pl.BlockSpec((T, H), lambda e, i: (0, 0)),
            pl.BlockSpec((1, BI, H), lambda e, i: (e, i, 0)),
            pl.BlockSpec((1, BI, H), lambda e, i: (e, i + NI, 0)),
            pl.BlockSpec((1, H, BI), lambda e, i: (e, 0, i)),
            pl.BlockSpec((T, E), lambda e, i: (0, 0)),
        ],
        out_specs=pl.BlockSpec((T, H), lambda e, i: (0, 0)),
        out_shape=jax.ShapeDtypeStruct((T, H), jnp.float32),
        compiler_params=pltpu.CompilerParams(
            dimension_semantics=("arbitrary", "arbitrary")),
    )(x_bf16, ws, ws, w2s, comb)


@jax.jit
def kernel(hidden_states, gate_weight, ws, w2s):
    scores = hidden_states @ gate_weight.T
    comb = _routing(scores)
    x_bf16 = hidden_states.astype(jnp.bfloat16)
    return _ffn(x_bf16, ws, w2s, comb)


# speedup vs baseline: 2.0179x; 1.0262x over previous
import functools

import jax
import jax.numpy as jnp
from jax.experimental import pallas as pl
from jax.experimental.pallas import tpu as pltpu

T = 1024
H = 2048
I = 2048
E = 8
JITTER_EPS = 0.01
BI = 512
NI = I // BI


def _routing_body(scores_ref, comb_ref):
    scores = scores_ref[...]
    col = jax.lax.broadcasted_iota(jnp.int32, scores.shape, 1)
    ninf = jnp.float32(-jnp.inf)

    m1 = jnp.max(scores, axis=1, keepdims=True)
    ind1 = jnp.min(jnp.where(scores == m1, col, E), axis=1, keepdims=True)
    factor1 = jnp.maximum(jnp.abs(scores), m1)
    mask1 = ((m1 - scores) / factor1) > (2.0 * JITTER_EPS)
    mg1 = jnp.where(mask1, ninf, scores)
    e1 = jnp.exp(mg1 - m1)
    p1 = e1 / jnp.sum(e1, axis=1, keepdims=True)
    mult1 = jnp.sum(jnp.where(col == ind1, p1, 0.0), axis=1, keepdims=True)

    masked_scores = jnp.where(col == ind1, ninf, scores)
    m2 = jnp.max(masked_scores, axis=1, keepdims=True)
    ind2 = jnp.min(jnp.where(masked_scores == m2, col, E), axis=1, keepdims=True)
    factor2 = jnp.maximum(jnp.abs(scores), m2)
    mask2 = ((m2 - scores) / factor2) > (2.0 * JITTER_EPS)
    mg2 = jnp.where(mask2, ninf, masked_scores)
    e2 = jnp.exp(mg2 - m2)
    p2 = e2 / jnp.sum(e2, axis=1, keepdims=True)
    mult2 = jnp.sum(jnp.where(col == ind2, p2, 0.0), axis=1, keepdims=True)

    comb_ref[...] = (jnp.where(col == ind1, mult1, 0.0)
                     + jnp.where(col == ind2, mult2, 0.0))


def _routing(scores):
    return pl.pallas_call(
        _routing_body,
        out_shape=jax.ShapeDtypeStruct((T, E), jnp.float32),
    )(scores)


def _ffn_body(x_ref, w1_ref, w3_ref, w2_ref, comb_ref, out_ref):
    e = pl.program_id(0)
    i = pl.program_id(1)

    @pl.when((e == 0) & (i == 0))
    def _init():
        out_ref[...] = jnp.zeros_like(out_ref)

    x = x_ref[...]
    w1 = w1_ref[0]
    w3 = w3_ref[0]
    h1 = jax.lax.dot_general(x, w1, (((1,), (1,)), ((), ())),
                             preferred_element_type=jnp.float32)
    h3 = jax.lax.dot_general(x, w3, (((1,), (1,)), ((), ())),
                             preferred_element_type=jnp.float32)
    act = h1 * jax.nn.sigmoid(h1) * h3
    w2 = w2_ref[0]
    y = jax.lax.dot_general(act, w2, (((1,), (1,)), ((), ())),
                            preferred_element_type=jnp.float32)
    comb = comb_ref[...]
    ecol = jax.lax.broadcasted_iota(jnp.int32, comb.shape, 1)
    cvec = jnp.sum(jnp.where(ecol == e, comb, 0.0), axis=1, keepdims=True)
    out_ref[...] += cvec * y


def _ffn(x_bf16, ws, w2s, comb):
    grid = (E, NI)
    return pl.pallas_call(
        _ffn_body,
        grid=grid,
        in_specs=[
            pl.BlockSpec((T, H), lambda e, i: (0, 0)),
            pl.BlockSpec((1, BI, H), lambda e, i: (e, i, 0)),
            pl.BlockSpec((1, BI, H), lambda e, i: (e, i + NI, 0)),
            pl.BlockSpec((1, H, BI), lambda e, i: (e, 0, i)),
            pl.BlockSpec((T, E), lambda e, i: (0, 0)),
        ],
        out_specs=pl.BlockSpec((T, H), lambda e, i: (0, 0)),
        out_shape=jax.ShapeDtypeStruct((T, H), jnp.float32),
        compiler_params=pltpu.CompilerParams(
            dimension_semantics=("arbitrary", "arbitrary")),
    )(x_bf16, ws, ws, w2s, comb)


@jax.jit
def kernel(hidden_states, gate_weight, ws, w2s):
    scores = hidden_states @ gate_weight.T
    comb = _routing(scores)
    return _ffn(hidden_states, ws, w2s, comb)
